# baseline (speedup 1.0000x reference)
import functools
import jax
import jax.numpy as jnp
from jax import lax
from jax.experimental import pallas as pl
from jax.experimental.pallas import tpu as pltpu
from jax.experimental.pallas import tpu_sc as plsc

VOCAB = 100000
EMBED = 10
CTX = 2
HIDDEN = 128
NW = 32
STRIPE = 3136
LAST_STRIPE = VOCAB - (NW - 1) * STRIPE
G_FULL = STRIPE // 16
G_LAST = LAST_STRIPE // 16
KB = HIDDEN // 8


def _mlp_body(x_ref, w1_ref, b1_ref, emb_hbm, hexp_ref, evmem, sem):
    for j in range(CTX):
        pltpu.make_async_copy(
            emb_hbm.at[pl.ds(x_ref[j], 1), :], evmem.at[pl.ds(j, 1), :],
            sem.at[j]).start()
    for j in range(CTX):
        pltpu.make_async_copy(
            emb_hbm.at[pl.ds(x_ref[j], 1), :], evmem.at[pl.ds(j, 1), :],
            sem.at[j]).wait()
    ht = jax.lax.dot_general(
        w1_ref[0], evmem[0:1, :], (((0,), (1,)), ((), ())),
        preferred_element_type=jnp.float32)
    ht = ht + jax.lax.dot_general(
        w1_ref[1], evmem[1:2, :], (((0,), (1,)), ((), ())),
        preferred_element_type=jnp.float32)
    ht = jnp.maximum(ht + b1_ref[...], 0.0)
    hexp_ref[...] = jnp.broadcast_to(ht, (HIDDEN, 16))


def _mlp(x, emb, W1, b1):
    return pl.pallas_call(
        _mlp_body,
        in_specs=[
            pl.BlockSpec(memory_space=pltpu.SMEM),
            pl.BlockSpec((CTX, EMBED, HIDDEN), lambda: (0, 0, 0)),
            pl.BlockSpec((HIDDEN, 1), lambda: (0, 0)),
            pl.BlockSpec(memory_space=pltpu.MemorySpace.HBM),
        ],
        out_specs=pl.BlockSpec((HIDDEN, 16), lambda: (0, 0)),
        out_shape=jax.ShapeDtypeStruct((HIDDEN, 16), jnp.float32),
        scratch_shapes=[
            pltpu.VMEM((CTX, EMBED), jnp.float32),
            pltpu.SemaphoreType.DMA((CTX,)),
        ],
    )(x, W1.reshape(CTX, EMBED, HIDDEN), b1.reshape(HIDDEN, 1), emb)


RING = 2
RBUF = 8 * STRIPE


def _sc_matvec(hexp1d, w2flat):
    mesh = plsc.VectorSubcoreMesh(core_axis_name="c", subcore_axis_name="s")

    @functools.partial(
        pl.kernel, mesh=mesh,
        out_type=jax.ShapeDtypeStruct((VOCAB,), jnp.float32),
        scratch_types=[
            pltpu.VMEM((2048,), jnp.float32),
            pltpu.VMEM((RING * RBUF,), jnp.float32),
            pltpu.VMEM((STRIPE,), jnp.float32),
            pltpu.SemaphoreType.DMA((RING,)),
            pltpu.SemaphoreType.DMA,
        ],
    )
    def k(h_hbm, w2_hbm, out_hbm, hbuf, ring, acc, sems, hsem):
        wid = lax.axis_index("s") * 2 + lax.axis_index("c")
        is_last = wid == NW - 1
        col0 = wid * STRIPE
        g_cnt = jnp.where(is_last, G_LAST, G_FULL)

        pltpu.make_async_copy(h_hbm, hbuf, hsem).start()

        def row_copies(b, slot, n):
            cs = []
            for kk in range(8):
                src = w2_hbm.at[pl.ds((8 * b + kk) * VOCAB + col0, n)]
                dst = ring.at[pl.ds(slot * RBUF + kk * STRIPE, n)]
                cs.append(pltpu.make_async_copy(src, dst, sems.at[slot]))
            return cs

        def start_block(b):
            slot = b % RING

            @pl.when(jnp.logical_not(is_last))
            def _():
                for c in row_copies(b, slot, STRIPE):
                    c.start()

            @pl.when(is_last)
            def _():
                for c in row_copies(b, slot, LAST_STRIPE):
                    c.start()

        def wait_block(b):
            slot = b % RING

            @pl.when(jnp.logical_not(is_last))
            def _():
                for c in row_copies(b, slot, STRIPE):
                    c.wait()

            @pl.when(is_last)
            def _():
                for c in row_copies(b, slot, LAST_STRIPE):
                    c.wait()

        for b in range(RING):
            start_block(b)
        pltpu.make_async_copy(h_hbm, hbuf, hsem).wait()

        for b in range(KB):
            slot = b % RING
            wait_block(b)
            hv = [hbuf[pl.ds((8 * b + kk) * 16, 16)] for kk in range(8)]

            def g_body(g, _):
                o = g * 16
                if b == 0:
                    a = jnp.zeros((16,), jnp.float32)
                else:
                    a = acc[pl.ds(o, 16)]
                for kk in range(8):
                    a = a + hv[kk] * ring[pl.ds(slot * RBUF + kk * STRIPE + o, 16)]
                acc[pl.ds(o, 16)] = a
                return 0

            lax.fori_loop(0, g_cnt, g_body, 0)
            if b + RING < KB:
                start_block(b + RING)

        @pl.when(jnp.logical_not(is_last))
        def _():
            pltpu.make_async_copy(
                acc.at[pl.ds(0, STRIPE)],
                out_hbm.at[pl.ds(col0, STRIPE)], hsem).start()
            pltpu.make_async_copy(
                acc.at[pl.ds(0, STRIPE)],
                out_hbm.at[pl.ds(col0, STRIPE)], hsem).wait()

        @pl.when(is_last)
        def _():
            pltpu.make_async_copy(
                acc.at[pl.ds(0, LAST_STRIPE)],
                out_hbm.at[pl.ds(col0, LAST_STRIPE)], hsem).start()
            pltpu.make_async_copy(
                acc.at[pl.ds(0, LAST_STRIPE)],
                out_hbm.at[pl.ds(col0, LAST_STRIPE)], hsem).wait()

    return k(hexp1d, w2flat)


def _finish_body(l_ref, b2_ref, out_ref):
    a = l_ref[...] + b2_ref[...]
    m = jnp.max(a, keepdims=True)
    s = jnp.sum(jnp.exp(a - m), keepdims=True)
    out_ref[...] = a - (m + jnp.log(s))


def _finish(logits1d, b2):
    return pl.pallas_call(
        _finish_body,
        in_specs=[
            pl.BlockSpec((VOCAB,), lambda: (0,)),
            pl.BlockSpec((VOCAB,), lambda: (0,)),
        ],
        out_specs=pl.BlockSpec((VOCAB,), lambda: (0,)),
        out_shape=jax.ShapeDtypeStruct((VOCAB,), jnp.float32),
    )(logits1d, b2)


def kernel(x, emb, W1, b1, W2, b2):
    hexp = _mlp(x, emb, W1, b1)
    logits1d = _sc_matvec(hexp.reshape(HIDDEN * 16), W2.reshape(HIDDEN * VOCAB))
    out1d = _finish(logits1d, b2)
    return out1d.reshape(1, VOCAB)


# speedup vs baseline: 1.1342x; 1.1342x over previous
import functools
import jax
import jax.numpy as jnp
from jax import lax
from jax.experimental import pallas as pl
from jax.experimental.pallas import tpu as pltpu
from jax.experimental.pallas import tpu_sc as plsc

VOCAB = 100000
EMBED = 10
CTX = 2
HIDDEN = 128
NW = 32
STRIPE = 3136
LAST_STRIPE = VOCAB - (NW - 1) * STRIPE
G_FULL = STRIPE // 16
G_LAST = LAST_STRIPE // 16
KB = HIDDEN // 8


def _mlp_body(x_ref, w1_ref, b1_ref, emb_hbm, hexp_ref, evmem, sem):
    for j in range(CTX):
        pltpu.make_async_copy(
            emb_hbm.at[pl.ds(x_ref[j], 1), :], evmem.at[pl.ds(j, 1), :],
            sem.at[j]).start()
    for j in range(CTX):
        pltpu.make_async_copy(
            emb_hbm.at[pl.ds(x_ref[j], 1), :], evmem.at[pl.ds(j, 1), :],
            sem.at[j]).wait()
    ht = jax.lax.dot_general(
        w1_ref[0], evmem[0:1, :], (((0,), (1,)), ((), ())),
        preferred_element_type=jnp.float32)
    ht = ht + jax.lax.dot_general(
        w1_ref[1], evmem[1:2, :], (((0,), (1,)), ((), ())),
        preferred_element_type=jnp.float32)
    ht = jnp.maximum(ht + b1_ref[...], 0.0)
    hexp_ref[...] = jnp.broadcast_to(ht, (HIDDEN, 16))


def _mlp(x, emb, W1, b1):
    return pl.pallas_call(
        _mlp_body,
        in_specs=[
            pl.BlockSpec(memory_space=pltpu.SMEM),
            pl.BlockSpec((CTX, EMBED, HIDDEN), lambda: (0, 0, 0)),
            pl.BlockSpec((HIDDEN, 1), lambda: (0, 0)),
            pl.BlockSpec(memory_space=pltpu.MemorySpace.HBM),
        ],
        out_specs=pl.BlockSpec((HIDDEN, 16), lambda: (0, 0)),
        out_shape=jax.ShapeDtypeStruct((HIDDEN, 16), jnp.float32),
        scratch_shapes=[
            pltpu.VMEM((CTX, EMBED), jnp.float32),
            pltpu.SemaphoreType.DMA((CTX,)),
        ],
    )(x, W1.reshape(CTX, EMBED, HIDDEN), b1.reshape(HIDDEN, 1), emb)


RING = 3
RBUF = 8 * STRIPE


def _sc_matvec(hexp1d, w2flat):
    mesh = plsc.VectorSubcoreMesh(core_axis_name="c", subcore_axis_name="s")

    @functools.partial(
        pl.kernel, mesh=mesh,
        compiler_params=pltpu.CompilerParams(use_tc_tiling_on_sc=False),
        out_type=jax.ShapeDtypeStruct((VOCAB,), jnp.float32),
        scratch_types=[
            pltpu.VMEM((2048,), jnp.float32),
            pltpu.VMEM((RING * RBUF,), jnp.float32),
            pltpu.VMEM((STRIPE,), jnp.float32),
            pltpu.SemaphoreType.DMA((RING,)),
            pltpu.SemaphoreType.DMA,
        ],
    )
    def k(h_hbm, w2_hbm, out_hbm, hbuf, ring, acc, sems, hsem):
        wid = lax.axis_index("s") * 2 + lax.axis_index("c")
        is_last = wid == NW - 1
        col0 = wid * STRIPE
        g_cnt = jnp.where(is_last, G_LAST, G_FULL)

        pltpu.make_async_copy(h_hbm, hbuf, hsem).start()

        def row_copies(b, slot, n):
            cs = []
            for kk in range(8):
                src = w2_hbm.at[8 * b + kk, pl.ds(col0, n)]
                dst = ring.at[pl.ds(slot * RBUF + kk * STRIPE, n)]
                cs.append(pltpu.make_async_copy(src, dst, sems.at[slot]))
            return cs

        def start_block(b):
            slot = b % RING

            @pl.when(jnp.logical_not(is_last))
            def _():
                for c in row_copies(b, slot, STRIPE):
                    c.start()

            @pl.when(is_last)
            def _():
                for c in row_copies(b, slot, LAST_STRIPE):
                    c.start()

        def wait_block(b):
            slot = b % RING

            @pl.when(jnp.logical_not(is_last))
            def _():
                for c in row_copies(b, slot, STRIPE):
                    c.wait()

            @pl.when(is_last)
            def _():
                for c in row_copies(b, slot, LAST_STRIPE):
                    c.wait()

        for b in range(RING):
            start_block(b)
        pltpu.make_async_copy(h_hbm, hbuf, hsem).wait()

        for b in range(KB):
            slot = b % RING
            wait_block(b)
            hv = [hbuf[pl.ds((8 * b + kk) * 16, 16)] for kk in range(8)]

            def g_body(g, _):
                o = g * 16
                if b == 0:
                    a = jnp.zeros((16,), jnp.float32)
                else:
                    a = acc[pl.ds(o, 16)]
                for kk in range(8):
                    a = a + hv[kk] * ring[pl.ds(slot * RBUF + kk * STRIPE + o, 16)]
                acc[pl.ds(o, 16)] = a
                return 0

            lax.fori_loop(0, G_LAST, g_body, 0, unroll=4)

            @pl.when(jnp.logical_not(is_last))
            def _():
                lax.fori_loop(G_LAST, G_FULL, g_body, 0, unroll=2)
            if b + RING < KB:
                start_block(b + RING)

        @pl.when(jnp.logical_not(is_last))
        def _():
            pltpu.make_async_copy(
                acc.at[pl.ds(0, STRIPE)],
                out_hbm.at[pl.ds(col0, STRIPE)], hsem).start()
            pltpu.make_async_copy(
                acc.at[pl.ds(0, STRIPE)],
                out_hbm.at[pl.ds(col0, STRIPE)], hsem).wait()

        @pl.when(is_last)
        def _():
            pltpu.make_async_copy(
                acc.at[pl.ds(0, LAST_STRIPE)],
                out_hbm.at[pl.ds(col0, LAST_STRIPE)], hsem).start()
            pltpu.make_async_copy(
                acc.at[pl.ds(0, LAST_STRIPE)],
                out_hbm.at[pl.ds(col0, LAST_STRIPE)], hsem).wait()

    return k(hexp1d, w2flat)


def _finish_body(l_ref, b2_ref, out_ref):
    a = l_ref[...] + b2_ref[...]
    m = jnp.max(a, keepdims=True)
    s = jnp.sum(jnp.exp(a - m), keepdims=True)
    out_ref[...] = a - (m + jnp.log(s))


def _finish(logits1d, b2):
    return pl.pallas_call(
        _finish_body,
        in_specs=[
            pl.BlockSpec((VOCAB,), lambda: (0,)),
            pl.BlockSpec((VOCAB,), lambda: (0,)),
        ],
        out_specs=pl.BlockSpec((VOCAB,), lambda: (0,)),
        out_shape=jax.ShapeDtypeStruct((VOCAB,), jnp.float32),
    )(logits1d, b2)


def kernel(x, emb, W1, b1, W2, b2):
    hexp = _mlp(x, emb, W1, b1)
    logits1d = _sc_matvec(hexp.reshape(HIDDEN * 16), W2)
    out1d = _finish(logits1d, b2)
    return out1d.reshape(1, VOCAB)


# speedup vs baseline: 2.1426x; 1.8891x over previous
import jax
import jax.numpy as jnp
from jax.experimental import pallas as pl
from jax.experimental.pallas import tpu as pltpu

VOCAB = 100000
EMBED = 10
CTX = 2
HIDDEN = 128
NB = HIDDEN // 8
NBUF = 4


def _mlp_body(x_ref, w1_ref, b1_ref, emb_hbm, hexp_ref, evmem, sem):
    for j in range(CTX):
        pltpu.make_async_copy(
            emb_hbm.at[pl.ds(x_ref[j], 1), :], evmem.at[pl.ds(j, 1), :],
            sem.at[j]).start()
    for j in range(CTX):
        pltpu.make_async_copy(
            emb_hbm.at[pl.ds(x_ref[j], 1), :], evmem.at[pl.ds(j, 1), :],
            sem.at[j]).wait()
    ht = jax.lax.dot_general(
        w1_ref[0], evmem[0:1, :], (((0,), (1,)), ((), ())),
        preferred_element_type=jnp.float32)
    ht = ht + jax.lax.dot_general(
        w1_ref[1], evmem[1:2, :], (((0,), (1,)), ((), ())),
        preferred_element_type=jnp.float32)
    ht = jnp.maximum(ht + b1_ref[...], 0.0)
    hexp_ref[...] = jnp.broadcast_to(ht, (HIDDEN, 16))


def _mlp(x, emb, W1, b1):
    return pl.pallas_call(
        _mlp_body,
        in_specs=[
            pl.BlockSpec(memory_space=pltpu.SMEM),
            pl.BlockSpec((CTX, EMBED, HIDDEN), lambda: (0, 0, 0)),
            pl.BlockSpec((HIDDEN, 1), lambda: (0, 0)),
            pl.BlockSpec(memory_space=pltpu.MemorySpace.HBM),
        ],
        out_specs=pl.BlockSpec((HIDDEN, 16), lambda: (0, 0)),
        out_shape=jax.ShapeDtypeStruct((HIDDEN, 16), jnp.float32),
        scratch_shapes=[
            pltpu.VMEM((CTX, EMBED), jnp.float32),
            pltpu.SemaphoreType.DMA((CTX,)),
        ],
    )(x, W1.reshape(CTX, EMBED, HIDDEN), b1.reshape(HIDDEN, 1), emb)


def _dense_body(hexp_ref, w2a_hbm, w2b_hbm, w2c_hbm, w2d_hbm, b2_ref,
                out_ref, acc_ref, buf0, buf1, buf2, buf3, sem_ref):
    srcs = (w2a_hbm, w2b_hbm, w2c_hbm, w2d_hbm)
    bufs = (buf0, buf1, buf2, buf3)

    def copy(b):
        q = b % NBUF
        return pltpu.make_async_copy(srcs[q].at[b], bufs[q], sem_ref.at[q])

    for b in range(NBUF):
        copy(b).start()

    for b in range(NB):
        copy(b).wait()
        hseg = hexp_ref[pl.ds(8 * b, 8), 0:1]
        partial = jax.lax.dot_general(
            hseg, bufs[b % NBUF][...], (((0,), (0,)), ((), ())),
            preferred_element_type=jnp.float32)
        if b == 0:
            acc_ref[...] = partial
        else:
            acc_ref[...] = acc_ref[...] + partial
        if b + NBUF < NB:
            copy(b + NBUF).start()

    a = acc_ref[...] + b2_ref[...]
    m = jnp.max(a, keepdims=True)
    s = jnp.sum(jnp.exp(a - m), keepdims=True)
    out_ref[...] = a - (m + jnp.log(s))


def _dense(hexp, W2, b2):
    w2v = W2.reshape(NB, 8, VOCAB)
    return pl.pallas_call(
        _dense_body,
        in_specs=[
            pl.BlockSpec((HIDDEN, 16), lambda: (0, 0)),
            pl.BlockSpec(memory_space=pltpu.MemorySpace.HBM),
            pl.BlockSpec(memory_space=pltpu.MemorySpace.HBM),
            pl.BlockSpec(memory_space=pltpu.MemorySpace.HBM),
            pl.BlockSpec(memory_space=pltpu.MemorySpace.HBM),
            pl.BlockSpec((1, VOCAB), lambda: (0, 0)),
        ],
        out_specs=pl.BlockSpec((1, VOCAB), lambda: (0, 0)),
        out_shape=jax.ShapeDtypeStruct((1, VOCAB), jnp.float32),
        scratch_shapes=[
            pltpu.VMEM((1, VOCAB), jnp.float32),
            pltpu.VMEM((8, VOCAB), jnp.float32),
            pltpu.VMEM((8, VOCAB), jnp.float32),
            pltpu.VMEM((8, VOCAB), jnp.float32),
            pltpu.VMEM((8, VOCAB), jnp.float32),
            pltpu.SemaphoreType.DMA((NBUF,)),
        ],
    )(hexp, w2v, w2v, w2v, w2v, b2.reshape(1, VOCAB))


def kernel(x, emb, W1, b1, W2, b2):
    hexp = _mlp(x, emb, W1, b1)
    return _dense(hexp, W2, b2)
